# baseline (speedup 1.0000x reference)
import jax
import jax.numpy as jnp
from jax import lax
from jax.experimental import pallas as pl
from jax.experimental.pallas import tpu as pltpu
from jax.experimental.pallas import tpu_sc as plsc

EMB = 64
BAG = 200
NCHUNK = EMB // 16


def _body(syms_hbm, table_hbm, out_hbm, idx_v, rows_v, out_v, sem):
    cid = lax.axis_index("c")
    sid = lax.axis_index("s")

    @pl.when((cid == 0) & (sid == 0))
    def _go():
        pltpu.sync_copy(syms_hbm, idx_v)
        pltpu.async_copy(table_hbm.at[idx_v], rows_v, sem).wait()
        for d in range(NCHUNK):
            accs = [rows_v[r, pl.ds(d * 16, 16)] for r in range(4)]
            for r in range(4, BAG, 4):
                for j in range(4):
                    accs[j] = accs[j] + rows_v[r + j, pl.ds(d * 16, 16)]
            out_v[pl.ds(d * 16, 16)] = (accs[0] + accs[1]) + (accs[2] + accs[3])
        pltpu.sync_copy(out_v, out_hbm)


@jax.jit
def _emb_sum(syms, table):
    mesh = plsc.VectorSubcoreMesh(
        core_axis_name="c", subcore_axis_name="s", num_cores=1)
    return pl.kernel(
        _body,
        out_type=jax.ShapeDtypeStruct((EMB,), jnp.float32),
        mesh=mesh,
        scratch_types=[
            pltpu.VMEM((BAG,), jnp.int32),
            pltpu.VMEM((BAG, EMB), jnp.float32),
            pltpu.VMEM((EMB,), jnp.float32),
            pltpu.SemaphoreType.DMA,
        ],
        compiler_params=pltpu.CompilerParams(use_tc_tiling_on_sc=False),
    )(syms, table)


def kernel(syms, table):
    return _emb_sum(syms.astype(jnp.int32), table)


# speedup vs baseline: 1.2068x; 1.2068x over previous
import jax
import jax.numpy as jnp
from jax import lax
from jax.experimental import pallas as pl
from jax.experimental.pallas import tpu as pltpu
from jax.experimental.pallas import tpu_sc as plsc

VOCAB = 1000
EMB = 64
BAG = 200
PER_W = 16
HALF = PER_W // 2
NFULL = BAG // PER_W
TAIL = BAG - NFULL * PER_W
NW = NFULL + 1
NCHUNK = EMB // 16


def _body(syms_hbm, table_hbm, out_hbm, idx_v, rows_v, part_v,
          parts_v, shared, sem_a, sem_b):
    cid = lax.axis_index("c")
    sid = lax.axis_index("s")

    @pl.when((cid == 0) & (sid < NFULL))
    def _full():
        base = sid * PER_W
        pltpu.sync_copy(syms_hbm.at[pl.ds(base, HALF)], idx_v.at[pl.ds(0, HALF)])
        cp_a = pltpu.async_copy(
            table_hbm.at[idx_v.at[pl.ds(0, HALF)]],
            rows_v.at[pl.ds(0, HALF)], sem_a)
        pltpu.sync_copy(syms_hbm.at[pl.ds(base + HALF, HALF)],
                        idx_v.at[pl.ds(HALF, HALF)])
        cp_b = pltpu.async_copy(
            table_hbm.at[idx_v.at[pl.ds(HALF, HALF)]],
            rows_v.at[pl.ds(HALF, HALF)], sem_b)
        cp_a.wait()
        accs = [None] * NCHUNK
        for d in range(NCHUNK):
            acc = rows_v[0, pl.ds(d * 16, 16)]
            for r in range(1, HALF):
                acc = acc + rows_v[r, pl.ds(d * 16, 16)]
            accs[d] = acc
        cp_b.wait()
        for d in range(NCHUNK):
            acc = accs[d]
            for r in range(HALF, PER_W):
                acc = acc + rows_v[r, pl.ds(d * 16, 16)]
            part_v[pl.ds(d * 16, 16)] = acc
        pltpu.sync_copy(part_v, shared.at[sid])

    @pl.when((cid == 0) & (sid == NFULL))
    def _tail():
        pltpu.sync_copy(syms_hbm.at[pl.ds(NFULL * PER_W, TAIL)],
                        idx_v.at[pl.ds(0, TAIL)])
        pltpu.async_copy(
            table_hbm.at[idx_v.at[pl.ds(0, TAIL)]],
            rows_v.at[pl.ds(0, TAIL)], sem_a).wait()
        for d in range(NCHUNK):
            acc = rows_v[0, pl.ds(d * 16, 16)]
            for r in range(1, TAIL):
                acc = acc + rows_v[r, pl.ds(d * 16, 16)]
            part_v[pl.ds(d * 16, 16)] = acc
        pltpu.sync_copy(part_v, shared.at[NFULL])

    plsc.subcore_barrier()

    @pl.when((cid == 0) & (sid < NCHUNK))
    def _reduce():
        pltpu.sync_copy(shared.at[:, pl.ds(sid * 16, 16)], parts_v)
        tot = parts_v[0, :]
        for r in range(1, NW):
            tot = tot + parts_v[r, :]
        part_v[pl.ds(0, 16)] = tot
        pltpu.sync_copy(part_v.at[pl.ds(0, 16)], out_hbm.at[pl.ds(sid * 16, 16)])


@jax.jit
def _emb_sum(syms, table):
    mesh = plsc.VectorSubcoreMesh(
        core_axis_name="c", subcore_axis_name="s", num_cores=1)
    return pl.kernel(
        _body,
        out_type=jax.ShapeDtypeStruct((EMB,), jnp.float32),
        mesh=mesh,
        scratch_types=[
            pltpu.VMEM((PER_W,), jnp.int32),
            pltpu.VMEM((PER_W, EMB), jnp.float32),
            pltpu.VMEM((EMB,), jnp.float32),
            pltpu.VMEM((NW, 16), jnp.float32),
            pltpu.VMEM_SHARED((NW, EMB), jnp.float32),
            pltpu.SemaphoreType.DMA,
            pltpu.SemaphoreType.DMA,
        ],
        compiler_params=pltpu.CompilerParams(use_tc_tiling_on_sc=False),
    )(syms, table)


def kernel(syms, table):
    return _emb_sum(syms.astype(jnp.int32), table)
